# baseline (speedup 1.0000x reference)
import functools

import jax
import jax.numpy as jnp
from jax import lax
from jax.experimental import pallas as pl
from jax.experimental.pallas import tpu as pltpu
from jax.experimental.pallas import tpu_sc as plsc

_B, _T, _H = 64, 512, 768
_N = _B * _T
_EPS = 1e-12

_S = 2
_BS = _B // _S
_NS_TOK = _N // _S

_NC, _NSC = 2, 16
_NW = _NC * _NSC
_RPW = _NS_TOK // _NW
_CHUNK = 32
_NCHUNK = _RPW // _CHUNK
_HP = _H // 2
_NVR = _HP // 16


def _sc_gather(word_table, ids):
    mesh = plsc.VectorSubcoreMesh(
        core_axis_name="c", subcore_axis_name="s",
        num_cores=_NC, num_subcores=_NSC)

    @functools.partial(
        pl.kernel,
        out_type=jax.ShapeDtypeStruct((_NS_TOK, _HP), jnp.int32),
        mesh=mesh,
        scratch_types=[
            pltpu.VMEM((_RPW,), jnp.int32),
            pltpu.VMEM((_CHUNK, _H), jnp.float32),
            pltpu.VMEM((_CHUNK, _H), jnp.float32),
            pltpu.VMEM((_CHUNK, _HP), jnp.int32),
            pltpu.VMEM((_CHUNK, _HP), jnp.int32),
            pltpu.SemaphoreType.DMA,
            pltpu.SemaphoreType.DMA,
            pltpu.SemaphoreType.DMA,
            pltpu.SemaphoreType.DMA,
        ],
        compiler_params=pltpu.CompilerParams(needs_layout_passes=False),
    )
    def k(word_hbm, ids_hbm, out_hbm, idx_v, rows0, rows1, pb0, pb1,
          g0, g1, w0, w1):
        wid = lax.axis_index("s") * _NC + lax.axis_index("c")
        base = wid * _RPW
        rows = (rows0, rows1)
        pbuf = (pb0, pb1)
        gsem = (g0, g1)
        wsem = (w0, w1)
        half = jnp.full((16,), 0x8000, jnp.int32)
        himask = jnp.full((16,), -65536, jnp.int32)
        pltpu.sync_copy(ids_hbm.at[pl.ds(base, _RPW)], idx_v)

        def gather(ch, b):
            return pltpu.async_copy(
                word_hbm.at[idx_v.at[pl.ds(ch * _CHUNK, _CHUNK)]],
                rows[b], gsem[b])

        def pack_rows(b):
            rv, pv = rows[b], pbuf[b]

            @plsc.parallel_loop(0, _CHUNK, unroll=4)
            def _rows(t):
                for j in range(_NVR):
                    a = plsc.bitcast(rv[t, pl.ds(16 * j, 16)], jnp.int32)
                    bb = plsc.bitcast(rv[t, pl.ds(_HP + 16 * j, 16)],
                                      jnp.int32)
                    lo = lax.shift_right_logical(a + half, 16)
                    hi = (bb + half) & himask
                    pv[t, pl.ds(16 * j, 16)] = lo | hi

        gather(0, 0)
        gather(1, 1)

        @pl.loop(0, _NCHUNK, step=2)
        def _ring(g):
            for b in range(2):
                ch = g + b
                pltpu.make_async_copy(
                    word_hbm.at[idx_v.at[pl.ds(ch * _CHUNK, _CHUNK)]],
                    rows[b], gsem[b]).wait()

                @pl.when(ch >= 2)
                def _():
                    pltpu.make_async_copy(
                        pbuf[b], out_hbm.at[pl.ds(base, _CHUNK)],
                        wsem[b]).wait()

                pack_rows(b)

                @pl.when(ch + 2 < _NCHUNK)
                def _():
                    gather(ch + 2, b)

                pltpu.async_copy(
                    pbuf[b], out_hbm.at[pl.ds(base + ch * _CHUNK, _CHUNK)],
                    wsem[b])

        for b in range(2):
            pltpu.make_async_copy(
                pbuf[b], out_hbm.at[pl.ds(base, _CHUNK)], wsem[b]).wait()

    return k(word_table, ids)


def _tc_body(wemb_ref, tt_ids_ref, pos_ref, tt_ref, g_ref, b_ref, out_ref):
    xu = wemb_ref[...]
    lo = lax.bitcast_convert_type(xu << 16, jnp.float32)
    hi = lax.bitcast_convert_type(xu & jnp.int32(-65536), jnp.float32)
    m = tt_ids_ref[...] == 1
    xl = lo + pos_ref[:, :_HP] + jnp.where(m, tt_ref[1:2, :_HP],
                                           tt_ref[0:1, :_HP])
    xh = hi + pos_ref[:, _HP:] + jnp.where(m, tt_ref[1:2, _HP:],
                                           tt_ref[0:1, _HP:])
    s = (jnp.sum(xl, axis=-1, keepdims=True)
         + jnp.sum(xh, axis=-1, keepdims=True))
    mean = s * (1.0 / _H)
    cl = xl - mean
    ch = xh - mean
    v = (jnp.sum(cl * cl, axis=-1, keepdims=True)
         + jnp.sum(ch * ch, axis=-1, keepdims=True)) * (1.0 / _H)
    r = lax.rsqrt(v + _EPS)
    out_ref[:, :_HP] = cl * r * g_ref[:, :_HP] + b_ref[:, :_HP]
    out_ref[:, _HP:] = ch * r * g_ref[:, _HP:] + b_ref[:, _HP:]


def _tc_body_acc(y_ref, wemb_ref, tt_ids_ref, pos_ref, tt_ref, g_ref, b_ref,
                 out_ref):
    del y_ref
    _tc_body(wemb_ref, tt_ids_ref, pos_ref, tt_ref, g_ref, b_ref, out_ref)


_RB = 512
_PB = _T // _RB

_DENSE_SPECS = [
    pl.BlockSpec((_RB, _HP), lambda i: (i, 0)),
    pl.BlockSpec((_RB, 1), lambda i: (i, 0)),
    pl.BlockSpec((_RB, _H), lambda i: (i % _PB, 0)),
    pl.BlockSpec((2, _H), lambda i: (0, 0)),
    pl.BlockSpec((1, _H), lambda i: (0, 0)),
    pl.BlockSpec((1, _H), lambda i: (0, 0)),
]


def _tc_add_ln_slice(y, wemb_s, tt_ids_s, pos, tt, g, b, s):
    nblk = _NS_TOK // _RB
    out_spec = pl.BlockSpec((_RB, _H), lambda i, s=s: (s * nblk + i, 0))
    if y is None:
        return pl.pallas_call(
            _tc_body,
            grid=(nblk,),
            in_specs=_DENSE_SPECS,
            out_specs=out_spec,
            out_shape=jax.ShapeDtypeStruct((_N, _H), jnp.float32),
        )(wemb_s, tt_ids_s, pos, tt, g, b)
    return pl.pallas_call(
        _tc_body_acc,
        grid=(nblk,),
        in_specs=[pl.BlockSpec(memory_space=pl.ANY)] + _DENSE_SPECS,
        out_specs=out_spec,
        out_shape=jax.ShapeDtypeStruct((_N, _H), jnp.float32),
        input_output_aliases={0: 0},
    )(y, wemb_s, tt_ids_s, pos, tt, g, b)


def kernel(input_ids, token_type_ids, word_table, pos_table, tt_table, gamma, beta):
    ids = input_ids.reshape(-1).astype(jnp.int32)
    tt_ids = token_type_ids.reshape(-1, 1).astype(jnp.int32)
    g = gamma.reshape(1, _H)
    b = beta.reshape(1, _H)

    wembs = [_sc_gather(word_table, ids[s * _NS_TOK:(s + 1) * _NS_TOK])
             for s in range(_S)]
    y = None
    for s in range(_S):
        tt_s = tt_ids[s * _NS_TOK:(s + 1) * _NS_TOK]
        y = _tc_add_ln_slice(y, wembs[s], tt_s, pos_table, tt_table, g, b, s)
    return y.reshape(_B, _T, _H)


# speedup vs baseline: 1.0001x; 1.0001x over previous
import functools

import jax
import jax.numpy as jnp
from jax import lax
from jax.experimental import pallas as pl
from jax.experimental.pallas import tpu as pltpu
from jax.experimental.pallas import tpu_sc as plsc

_B, _T, _H = 64, 512, 768
_N = _B * _T
_EPS = 1e-12

_S = 4
_BS = _B // _S
_NS_TOK = _N // _S

_NC, _NSC = 2, 16
_NW = _NC * _NSC
_RPW = _NS_TOK // _NW
_CHUNK = 32
_NCHUNK = _RPW // _CHUNK
_HP = _H // 2
_NVR = _HP // 16


def _sc_gather(word_table, ids):
    mesh = plsc.VectorSubcoreMesh(
        core_axis_name="c", subcore_axis_name="s",
        num_cores=_NC, num_subcores=_NSC)

    @functools.partial(
        pl.kernel,
        out_type=jax.ShapeDtypeStruct((_NS_TOK, _HP), jnp.int32),
        mesh=mesh,
        scratch_types=[
            pltpu.VMEM((_RPW,), jnp.int32),
            pltpu.VMEM((_CHUNK, _H), jnp.float32),
            pltpu.VMEM((_CHUNK, _H), jnp.float32),
            pltpu.VMEM((_CHUNK, _HP), jnp.int32),
            pltpu.VMEM((_CHUNK, _HP), jnp.int32),
            pltpu.SemaphoreType.DMA,
            pltpu.SemaphoreType.DMA,
            pltpu.SemaphoreType.DMA,
            pltpu.SemaphoreType.DMA,
        ],
        compiler_params=pltpu.CompilerParams(needs_layout_passes=False),
    )
    def k(word_hbm, ids_hbm, out_hbm, idx_v, rows0, rows1, pb0, pb1,
          g0, g1, w0, w1):
        wid = lax.axis_index("s") * _NC + lax.axis_index("c")
        base = wid * _RPW
        rows = (rows0, rows1)
        pbuf = (pb0, pb1)
        gsem = (g0, g1)
        wsem = (w0, w1)
        half = jnp.full((16,), 0x8000, jnp.int32)
        himask = jnp.full((16,), -65536, jnp.int32)
        pltpu.sync_copy(ids_hbm.at[pl.ds(base, _RPW)], idx_v)

        def gather(ch, b):
            return pltpu.async_copy(
                word_hbm.at[idx_v.at[pl.ds(ch * _CHUNK, _CHUNK)]],
                rows[b], gsem[b])

        def pack_rows(b):
            rv, pv = rows[b], pbuf[b]

            @plsc.parallel_loop(0, _CHUNK, unroll=4)
            def _rows(t):
                for j in range(_NVR):
                    a = plsc.bitcast(rv[t, pl.ds(16 * j, 16)], jnp.int32)
                    bb = plsc.bitcast(rv[t, pl.ds(_HP + 16 * j, 16)],
                                      jnp.int32)
                    lo = lax.shift_right_logical(a + half, 16)
                    hi = (bb + half) & himask
                    pv[t, pl.ds(16 * j, 16)] = lo | hi

        gather(0, 0)
        gather(1, 1)

        @pl.loop(0, _NCHUNK, step=2)
        def _ring(g):
            for b in range(2):
                ch = g + b
                pltpu.make_async_copy(
                    word_hbm.at[idx_v.at[pl.ds(ch * _CHUNK, _CHUNK)]],
                    rows[b], gsem[b]).wait()

                @pl.when(ch >= 2)
                def _():
                    pltpu.make_async_copy(
                        pbuf[b], out_hbm.at[pl.ds(base, _CHUNK)],
                        wsem[b]).wait()

                pack_rows(b)

                @pl.when(ch + 2 < _NCHUNK)
                def _():
                    gather(ch + 2, b)

                pltpu.async_copy(
                    pbuf[b], out_hbm.at[pl.ds(base + ch * _CHUNK, _CHUNK)],
                    wsem[b])

        for b in range(2):
            pltpu.make_async_copy(
                pbuf[b], out_hbm.at[pl.ds(base, _CHUNK)], wsem[b]).wait()

    return k(word_table, ids)


def _tc_body(wemb_ref, tt_ids_ref, pos_ref, tt_ref, g_ref, b_ref, out_ref):
    xu = wemb_ref[...]
    lo = lax.bitcast_convert_type(xu << 16, jnp.float32)
    hi = lax.bitcast_convert_type(xu & jnp.int32(-65536), jnp.float32)
    m = tt_ids_ref[...] == 1
    xl = lo + pos_ref[:, :_HP] + jnp.where(m, tt_ref[1:2, :_HP],
                                           tt_ref[0:1, :_HP])
    xh = hi + pos_ref[:, _HP:] + jnp.where(m, tt_ref[1:2, _HP:],
                                           tt_ref[0:1, _HP:])
    s = (jnp.sum(xl, axis=-1, keepdims=True)
         + jnp.sum(xh, axis=-1, keepdims=True))
    mean = s * (1.0 / _H)
    cl = xl - mean
    ch = xh - mean
    v = (jnp.sum(cl * cl, axis=-1, keepdims=True)
         + jnp.sum(ch * ch, axis=-1, keepdims=True)) * (1.0 / _H)
    r = lax.rsqrt(v + _EPS)
    out_ref[:, :_HP] = cl * r * g_ref[:, :_HP] + b_ref[:, :_HP]
    out_ref[:, _HP:] = ch * r * g_ref[:, _HP:] + b_ref[:, _HP:]


def _tc_body_acc(y_ref, wemb_ref, tt_ids_ref, pos_ref, tt_ref, g_ref, b_ref,
                 out_ref):
    del y_ref
    _tc_body(wemb_ref, tt_ids_ref, pos_ref, tt_ref, g_ref, b_ref, out_ref)


_RB = 512
_PB = _T // _RB

_DENSE_SPECS = [
    pl.BlockSpec((_RB, _HP), lambda i: (i, 0)),
    pl.BlockSpec((_RB, 1), lambda i: (i, 0)),
    pl.BlockSpec((_RB, _H), lambda i: (i % _PB, 0)),
    pl.BlockSpec((2, _H), lambda i: (0, 0)),
    pl.BlockSpec((1, _H), lambda i: (0, 0)),
    pl.BlockSpec((1, _H), lambda i: (0, 0)),
]


def _tc_add_ln_slice(y, wemb_s, tt_ids_s, pos, tt, g, b, s):
    nblk = _NS_TOK // _RB
    out_spec = pl.BlockSpec((_RB, _H), lambda i, s=s: (s * nblk + i, 0))
    if y is None:
        return pl.pallas_call(
            _tc_body,
            grid=(nblk,),
            in_specs=_DENSE_SPECS,
            out_specs=out_spec,
            out_shape=jax.ShapeDtypeStruct((_N, _H), jnp.float32),
        )(wemb_s, tt_ids_s, pos, tt, g, b)
    return pl.pallas_call(
        _tc_body_acc,
        grid=(nblk,),
        in_specs=[pl.BlockSpec(memory_space=pl.ANY)] + _DENSE_SPECS,
        out_specs=out_spec,
        out_shape=jax.ShapeDtypeStruct((_N, _H), jnp.float32),
        input_output_aliases={0: 0},
    )(y, wemb_s, tt_ids_s, pos, tt, g, b)


def kernel(input_ids, token_type_ids, word_table, pos_table, tt_table, gamma, beta):
    ids = input_ids.reshape(-1).astype(jnp.int32)
    tt_ids = token_type_ids.reshape(-1, 1).astype(jnp.int32)
    g = gamma.reshape(1, _H)
    b = beta.reshape(1, _H)

    wembs = [_sc_gather(word_table, ids[s * _NS_TOK:(s + 1) * _NS_TOK])
             for s in range(_S)]
    y = None
    for s in range(_S):
        tt_s = tt_ids[s * _NS_TOK:(s + 1) * _NS_TOK]
        y = _tc_add_ln_slice(y, wembs[s], tt_s, pos_table, tt_table, g, b, s)
    return y.reshape(_B, _T, _H)


# speedup vs baseline: 1.0989x; 1.0988x over previous
import functools

import jax
import jax.numpy as jnp
from jax import lax
from jax.experimental import pallas as pl
from jax.experimental.pallas import tpu as pltpu
from jax.experimental.pallas import tpu_sc as plsc

_B, _T, _H = 64, 512, 768
_N = _B * _T
_EPS = 1e-12

_S = 2
_BS = _B // _S
_NS_TOK = _N // _S

_NC, _NSC = 2, 16
_NW = _NC * _NSC
_RPW = _NS_TOK // _NW
_CHUNK = 32
_NCHUNK = _RPW // _CHUNK
_HP = _H // 2
_NVR = _HP // 16


def _sc_gather(word_table, ids, tts, s_off):
    mesh = plsc.VectorSubcoreMesh(
        core_axis_name="c", subcore_axis_name="s",
        num_cores=_NC, num_subcores=_NSC)

    @functools.partial(
        pl.kernel,
        out_type=jax.ShapeDtypeStruct((_NS_TOK, _HP), jnp.int32),
        mesh=mesh,
        scratch_types=[
            pltpu.VMEM((_RPW,), jnp.int32),
            pltpu.VMEM((_RPW,), jnp.int32),
            pltpu.VMEM((_CHUNK, _H), jnp.float32),
            pltpu.VMEM((_CHUNK, _H), jnp.float32),
            pltpu.VMEM((_CHUNK, _HP), jnp.int32),
            pltpu.VMEM((_CHUNK, _HP), jnp.int32),
            pltpu.SemaphoreType.DMA,
            pltpu.SemaphoreType.DMA,
            pltpu.SemaphoreType.DMA,
            pltpu.SemaphoreType.DMA,
        ],
        compiler_params=pltpu.CompilerParams(needs_layout_passes=False),
    )
    def k(word_hbm, ids_hbm, tts_hbm, out_hbm, idx_v, tts_v, rows0, rows1,
          pb0, pb1, g0, g1, w0, w1):
        wid = lax.axis_index("s") * _NC + lax.axis_index("c")
        base = wid * _RPW
        rows = (rows0, rows1)
        pbuf = (pb0, pb1)
        gsem = (g0, g1)
        wsem = (w0, w1)
        half = jnp.full((16,), 0x8000, jnp.int32)
        himask = jnp.full((16,), -65536, jnp.int32)
        one_v = jnp.full((16,), 1, jnp.int32)
        lane = lax.iota(jnp.int32, 16)
        lane0mask = jnp.where(lane == jnp.full((16,), 0, jnp.int32),
                              jnp.full((16,), -2, jnp.int32),
                              jnp.full((16,), -1, jnp.int32))
        col0 = jnp.full((16,), 0, jnp.int32)
        pltpu.sync_copy(ids_hbm.at[pl.ds(s_off + base, _RPW)], idx_v)
        pltpu.sync_copy(tts_hbm.at[pl.ds(s_off + base, _RPW)], tts_v)

        def gather(ch, b):
            return pltpu.async_copy(
                word_hbm.at[idx_v.at[pl.ds(ch * _CHUNK, _CHUNK)]],
                rows[b], gsem[b])

        def pack_rows(ch, b):
            rv, pv = rows[b], pbuf[b]

            @plsc.parallel_loop(0, _CHUNK, unroll=4)
            def _rows(t):
                for j in range(_NVR):
                    a = plsc.bitcast(rv[t, pl.ds(16 * j, 16)], jnp.int32)
                    bb = plsc.bitcast(rv[t, pl.ds(_HP + 16 * j, 16)],
                                      jnp.int32)
                    lo = lax.shift_right_logical(a + half, 16)
                    hi = (bb + half) & himask
                    w = lo | hi
                    if j == 0:
                        w = w & lane0mask
                    pv[t, pl.ds(16 * j, 16)] = w

            for h in range(_CHUNK // 16):
                rowidx = lane + jnp.full((16,), 16 * h, jnp.int32)
                tt16 = tts_v[pl.ds(ch * _CHUNK + 16 * h, 16)] & one_v
                cur = plsc.load_gather(pv, [rowidx, col0])
                plsc.store_scatter(pv, [rowidx, col0], cur | tt16)

        gather(0, 0)
        gather(1, 1)

        @pl.loop(0, _NCHUNK, step=2)
        def _ring(g):
            for b in range(2):
                ch = g + b
                pltpu.make_async_copy(
                    word_hbm.at[idx_v.at[pl.ds(ch * _CHUNK, _CHUNK)]],
                    rows[b], gsem[b]).wait()

                @pl.when(ch >= 2)
                def _():
                    pltpu.make_async_copy(
                        pbuf[b], out_hbm.at[pl.ds(base, _CHUNK)],
                        wsem[b]).wait()

                pack_rows(ch, b)

                @pl.when(ch + 2 < _NCHUNK)
                def _():
                    gather(ch + 2, b)

                pltpu.async_copy(
                    pbuf[b], out_hbm.at[pl.ds(base + ch * _CHUNK, _CHUNK)],
                    wsem[b])

        for b in range(2):
            pltpu.make_async_copy(
                pbuf[b], out_hbm.at[pl.ds(base, _CHUNK)], wsem[b]).wait()

    return k(word_table, ids, tts)


def _tc_body(wemb_ref, pos_ref, tt_ref, g_ref, b_ref, out_ref):
    xu = wemb_ref[...]
    lo = lax.bitcast_convert_type(xu << 16, jnp.float32)
    hi = lax.bitcast_convert_type(xu & jnp.int32(-65536), jnp.float32)
    m = (xu[:, 0:1] & 1) == 1
    xl = lo + pos_ref[:, :_HP] + jnp.where(m, tt_ref[1:2, :_HP],
                                           tt_ref[0:1, :_HP])
    xh = hi + pos_ref[:, _HP:] + jnp.where(m, tt_ref[1:2, _HP:],
                                           tt_ref[0:1, _HP:])
    s = (jnp.sum(xl, axis=-1, keepdims=True)
         + jnp.sum(xh, axis=-1, keepdims=True))
    mean = s * (1.0 / _H)
    cl = xl - mean
    ch = xh - mean
    v = (jnp.sum(cl * cl, axis=-1, keepdims=True)
         + jnp.sum(ch * ch, axis=-1, keepdims=True)) * (1.0 / _H)
    r = lax.rsqrt(v + _EPS)
    out_ref[:, :_HP] = cl * r * g_ref[:, :_HP] + b_ref[:, :_HP]
    out_ref[:, _HP:] = ch * r * g_ref[:, _HP:] + b_ref[:, _HP:]


def _tc_body_acc(y_ref, wemb_ref, pos_ref, tt_ref, g_ref, b_ref, out_ref):
    del y_ref
    _tc_body(wemb_ref, pos_ref, tt_ref, g_ref, b_ref, out_ref)


_RB = 512
_PB = _T // _RB

_DENSE_SPECS = [
    pl.BlockSpec((_RB, _HP), lambda i: (i, 0)),
    pl.BlockSpec((_RB, _H), lambda i: (i % _PB, 0)),
    pl.BlockSpec((2, _H), lambda i: (0, 0)),
    pl.BlockSpec((1, _H), lambda i: (0, 0)),
    pl.BlockSpec((1, _H), lambda i: (0, 0)),
]


def _tc_add_ln_slice(y, wemb_s, pos, tt, g, b, s):
    nblk = _NS_TOK // _RB
    out_spec = pl.BlockSpec((_RB, _H), lambda i, s=s: (s * nblk + i, 0))
    if y is None:
        return pl.pallas_call(
            _tc_body,
            grid=(nblk,),
            in_specs=_DENSE_SPECS,
            out_specs=out_spec,
            out_shape=jax.ShapeDtypeStruct((_N, _H), jnp.float32),
        )(wemb_s, pos, tt, g, b)
    return pl.pallas_call(
        _tc_body_acc,
        grid=(nblk,),
        in_specs=[pl.BlockSpec(memory_space=pl.ANY)] + _DENSE_SPECS,
        out_specs=out_spec,
        out_shape=jax.ShapeDtypeStruct((_N, _H), jnp.float32),
        input_output_aliases={0: 0},
    )(y, wemb_s, pos, tt, g, b)


def kernel(input_ids, token_type_ids, word_table, pos_table, tt_table, gamma, beta):
    ids = input_ids.reshape(-1).astype(jnp.int32)
    tts = token_type_ids.reshape(-1).astype(jnp.int32)
    g = gamma.reshape(1, _H)
    b = beta.reshape(1, _H)

    wembs = [_sc_gather(word_table, ids, tts, s * _NS_TOK)
             for s in range(_S)]
    y = None
    for s in range(_S):
        y = _tc_add_ln_slice(y, wembs[s], pos_table, tt_table, g, b, s)
    return y.reshape(_B, _T, _H)
